# Initial kernel scaffold; baseline (speedup 1.0000x reference)
import jax
import jax.numpy as jnp
from jax.experimental import pallas as pl


def kernel(token_ids, weight):
    raise NotImplementedError("write your pallas kernel here")


# speedup vs baseline: 1.7937x; 1.7937x over previous
import functools

import jax
import jax.numpy as jnp
from jax import lax
from jax.experimental import pallas as pl
from jax.experimental.pallas import tpu as pltpu
from jax.experimental.pallas import tpu_sc as plsc

_B_TOK = 16384
_T = 50
_D = 64
_B = _B_TOK * _T
_NC = 2
_NS = 16
_NW = _NC * _NS
_B_PER_W = _B // _NW
_CHUNK = 512
_N_CHUNKS = _B_PER_W // _CHUNK

_mesh = plsc.VectorSubcoreMesh(core_axis_name="c", subcore_axis_name="s")


@functools.partial(
    pl.kernel,
    mesh=_mesh,
    out_type=jax.ShapeDtypeStruct((_B, _D), jnp.float32),
    scratch_types=[
        pltpu.VMEM((_CHUNK,), jnp.int32),
        pltpu.VMEM((_CHUNK, _D), jnp.float32),
        pltpu.SemaphoreType.DMA,
    ],
    compiler_params=pltpu.CompilerParams(use_tc_tiling_on_sc=False),
)
def _gather_rows(idx_hbm, table_hbm, out_hbm, idx_v, rows_v, sem):
    wid = lax.axis_index("s") * _NC + lax.axis_index("c")
    base = wid * _B_PER_W

    @pl.loop(0, _N_CHUNKS)
    def _chunk(i):
        off = base + i * _CHUNK
        pltpu.sync_copy(idx_hbm.at[pl.ds(off, _CHUNK)], idx_v)
        pltpu.async_copy(table_hbm.at[idx_v], rows_v, sem).wait()
        pltpu.sync_copy(rows_v, out_hbm.at[pl.ds(off, _CHUNK)])


def kernel(token_ids, weight):
    idx = token_ids.reshape(_B).astype(jnp.int32)
    out = _gather_rows(idx, weight)
    return out.reshape(_B_TOK, _T, _D)


# speedup vs baseline: 1.8751x; 1.0454x over previous
import functools

import jax
import jax.numpy as jnp
from jax import lax
from jax.experimental import pallas as pl
from jax.experimental.pallas import tpu as pltpu
from jax.experimental.pallas import tpu_sc as plsc

_B_TOK = 16384
_T = 50
_D = 64
_B = _B_TOK * _T
_NC = 2
_NS = 16
_NW = _NC * _NS
_B_PER_W = _B // _NW
_CHUNK = 512
_N_CHUNKS = _B_PER_W // _CHUNK

_mesh = plsc.VectorSubcoreMesh(core_axis_name="c", subcore_axis_name="s")


@functools.partial(
    pl.kernel,
    mesh=_mesh,
    out_type=jax.ShapeDtypeStruct((_B, _D), jnp.float32),
    scratch_types=[
        pltpu.VMEM((_CHUNK,), jnp.int32),
        pltpu.VMEM((_CHUNK,), jnp.int32),
        pltpu.VMEM((_CHUNK, _D), jnp.float32),
        pltpu.VMEM((_CHUNK, _D), jnp.float32),
        pltpu.SemaphoreType.DMA,
        pltpu.SemaphoreType.DMA,
        pltpu.SemaphoreType.DMA,
        pltpu.SemaphoreType.DMA,
    ],
    compiler_params=pltpu.CompilerParams(use_tc_tiling_on_sc=False),
)
def _gather_rows(idx_hbm, table_hbm, out_hbm,
                 idx0, idx1, rows0, rows1, gs0, gs1, ss0, ss1):
    wid = lax.axis_index("s") * _NC + lax.axis_index("c")
    base = wid * _B_PER_W
    bufs = ((idx0, rows0, gs0, ss0), (idx1, rows1, gs1, ss1))

    def load_idx(c, idx_v):
        pltpu.sync_copy(idx_hbm.at[pl.ds(base + c * _CHUNK, _CHUNK)], idx_v)

    def fire_gather(idx_v, rows_v, sem):
        pltpu.async_copy(table_hbm.at[idx_v], rows_v, sem)

    def wait_gather(idx_v, rows_v, sem):
        pltpu.make_async_copy(table_hbm.at[idx_v], rows_v, sem).wait()

    def fire_store(c, rows_v, sem):
        return pltpu.async_copy(
            rows_v, out_hbm.at[pl.ds(base + c * _CHUNK, _CHUNK)], sem)

    for b, (idx_v, rows_v, gsem, _) in enumerate(bufs):
        load_idx(b, idx_v)
        fire_gather(idx_v, rows_v, gsem)

    @pl.loop(0, _N_CHUNKS - 2, step=2)
    def _pair(i):
        for b, (idx_v, rows_v, gsem, ssem) in enumerate(bufs):
            c = i + b
            wait_gather(idx_v, rows_v, gsem)
            store = fire_store(c, rows_v, ssem)
            load_idx(c + 2, idx_v)
            store.wait()
            fire_gather(idx_v, rows_v, gsem)

    stores = []
    for b, (idx_v, rows_v, gsem, ssem) in enumerate(bufs):
        wait_gather(idx_v, rows_v, gsem)
        stores.append(fire_store(_N_CHUNKS - 2 + b, rows_v, ssem))
    for store in stores:
        store.wait()


def kernel(token_ids, weight):
    idx = token_ids.reshape(_B).astype(jnp.int32)
    out = _gather_rows(idx, weight)
    return out.reshape(_B_TOK, _T, _D)


# speedup vs baseline: 1.8843x; 1.0049x over previous
import functools

import jax
import jax.numpy as jnp
from jax import lax
from jax.experimental import pallas as pl
from jax.experimental.pallas import tpu as pltpu
from jax.experimental.pallas import tpu_sc as plsc

_B_TOK = 16384
_T = 50
_D = 64
_B = _B_TOK * _T
_NC = 2
_NS = 16
_NW = _NC * _NS
_B_PER_W = _B // _NW
_CHUNK = 512
_N_CHUNKS = _B_PER_W // _CHUNK

_mesh = plsc.VectorSubcoreMesh(core_axis_name="c", subcore_axis_name="s")


@functools.partial(
    pl.kernel,
    mesh=_mesh,
    out_type=jax.ShapeDtypeStruct((_B_TOK, _T, _D), jnp.float32),
    scratch_types=[
        pltpu.VMEM((_CHUNK,), jnp.int32),
        pltpu.VMEM((_CHUNK,), jnp.int32),
        pltpu.VMEM((_CHUNK, _D), jnp.float32),
        pltpu.VMEM((_CHUNK, _D), jnp.float32),
        pltpu.SemaphoreType.DMA,
        pltpu.SemaphoreType.DMA,
        pltpu.SemaphoreType.DMA,
        pltpu.SemaphoreType.DMA,
    ],
    compiler_params=pltpu.CompilerParams(use_tc_tiling_on_sc=False),
)
def _gather_rows(idx_hbm, table_hbm, out_hbm,
                 idx0, idx1, rows0, rows1, gs0, gs1, ss0, ss1):
    wid = lax.axis_index("s") * _NC + lax.axis_index("c")
    base = wid * _B_PER_W
    bufs = ((idx0, rows0, gs0, ss0), (idx1, rows1, gs1, ss1))

    def load_idx(c, idx_v):
        pltpu.sync_copy(idx_hbm.at[pl.ds(base + c * _CHUNK, _CHUNK)], idx_v)

    def fire_gather(idx_v, rows_v, sem):
        pltpu.async_copy(table_hbm.at[idx_v], rows_v, sem)

    def wait_gather(idx_v, rows_v, sem):
        pltpu.make_async_copy(table_hbm.at[idx_v], rows_v, sem).wait()

    def fire_store(c, rows_v, sem):
        fl = base + c * _CHUNK
        t = fl // _B_TOK
        b0 = fl % _B_TOK
        return pltpu.async_copy(rows_v, out_hbm.at[pl.ds(b0, _CHUNK), t], sem)

    for b, (idx_v, rows_v, gsem, _) in enumerate(bufs):
        load_idx(b, idx_v)
        fire_gather(idx_v, rows_v, gsem)

    @pl.loop(0, _N_CHUNKS - 2, step=2)
    def _pair(i):
        for b, (idx_v, rows_v, gsem, ssem) in enumerate(bufs):
            c = i + b
            wait_gather(idx_v, rows_v, gsem)
            store = fire_store(c, rows_v, ssem)
            load_idx(c + 2, idx_v)
            store.wait()
            fire_gather(idx_v, rows_v, gsem)

    stores = []
    for b, (idx_v, rows_v, gsem, ssem) in enumerate(bufs):
        wait_gather(idx_v, rows_v, gsem)
        stores.append(fire_store(_N_CHUNKS - 2 + b, rows_v, ssem))
    for store in stores:
        store.wait()


def kernel(token_ids, weight):
    idx = token_ids.T.ravel().astype(jnp.int32)
    return _gather_rows(idx, weight)
